# baseline (speedup 1.0000x reference)
import functools

import jax
import jax.numpy as jnp
from jax import lax
from jax.experimental import pallas as pl
from jax.experimental.pallas import tpu as pltpu
from jax.experimental.pallas import tpu_sc as plsc

N = 10000
E = 160000
INPUT_DIM = 128
DIM = 16
WIDTH = 64
B = 64
EDGE_DIM = 5

NC = 2
NS = 16
NW = NC * NS

EPW = 5120
E_PAD = NW * EPW
CHUNK = 128
NCHUNK = EPW // CHUNK
EPS = E_PAD // NS
GPASS = EPS // EPW
N_PAD = 10240
ROWS_PER_TILE = N_PAD // NS
DUMMY_ROW = N

TILE_E = 2048


def _node_mlp_body(feats_ref, w_ref, b_ref, out_ref):
    acc = jnp.dot(feats_ref[...], w_ref[...], preferred_element_type=jnp.float32)
    out_ref[...] = jnp.maximum(acc + b_ref[...], 0.0)


def _node_mlp(feats, w0, b0):
    return pl.pallas_call(
        _node_mlp_body,
        out_shape=jax.ShapeDtypeStruct((N, DIM), jnp.float32),
    )(feats, w0, b0.reshape(1, DIM))


def _we_msg(ea, wn1, bn1, wn2, bn2, s):
    hidden = jnp.dot(ea, wn1, preferred_element_type=jnp.float32,
                     precision=lax.Precision.HIGHEST)
    hidden = jnp.maximum(hidden + bn1, 0.0)
    we = jnp.dot(hidden, wn2, preferred_element_type=jnp.float32,
                 precision=lax.Precision.HIGHEST)
    we = we + bn2
    acc = s[:, 0:1] * we[:, 0:DIM]
    for d in range(1, DIM):
        acc = acc + s[:, d:d + 1] * we[:, DIM * d:DIM * (d + 1)]
    return acc


def _matvec_body(g_ref, ea_ref, wn1_ref, bn1_ref, wn2_ref, bn2_ref, msg_ref):
    msg_ref[...] = _we_msg(ea_ref[...], wn1_ref[...], bn1_ref[...],
                           wn2_ref[...], bn2_ref[...], g_ref[...])


def _matvec(g, ea_pad, wn1, bn1, wn2, bn2):
    grid = (E_PAD // TILE_E,)
    return pl.pallas_call(
        _matvec_body,
        grid=grid,
        in_specs=[
            pl.BlockSpec((TILE_E, DIM), lambda i: (i, 0)),
            pl.BlockSpec((TILE_E, EDGE_DIM), lambda i: (i, 0)),
            pl.BlockSpec((EDGE_DIM, WIDTH), lambda i: (0, 0)),
            pl.BlockSpec((1, WIDTH), lambda i: (0, 0)),
            pl.BlockSpec((WIDTH, DIM * DIM), lambda i: (0, 0)),
            pl.BlockSpec((1, DIM * DIM), lambda i: (0, 0)),
        ],
        out_specs=pl.BlockSpec((TILE_E, DIM), lambda i: (i, 0)),
        out_shape=jax.ShapeDtypeStruct((E_PAD, DIM), jnp.float32),
    )(g, ea_pad, wn1, bn1.reshape(1, WIDTH), wn2, bn2.reshape(1, DIM * DIM))


def _gru_math(s, agg, wroot, bconv, wihT, whhT, bih, bhh):
    m = jnp.dot(s, wroot, preferred_element_type=jnp.float32)
    m = jnp.maximum(m + agg + bconv, 0.0)
    gi = jnp.dot(m, wihT, preferred_element_type=jnp.float32) + bih
    gh = jnp.dot(s, whhT, preferred_element_type=jnp.float32) + bhh
    r = jax.nn.sigmoid(gi[:, :DIM] + gh[:, :DIM])
    u = jax.nn.sigmoid(gi[:, DIM:2 * DIM] + gh[:, DIM:2 * DIM])
    n = jnp.tanh(gi[:, 2 * DIM:] + r * gh[:, 2 * DIM:])
    return (1.0 - u) * n + u * s


def _edge_step_body(with_cnt, *refs):
    if with_cnt:
        (s_ref, gap_ref, gcp_ref, ea_ref,
         wn1_ref, bn1_ref, wn2_ref, bn2_ref, wroot_ref, bconv_ref,
         wihT_ref, whhT_ref, bih_ref, bhh_ref,
         snew_ref, msg_ref, gc_ref) = refs
        gcp = gcp_ref[...]
        gc = jnp.maximum(gcp[0] + gcp[1], 1.0)
        gc_ref[...] = gc
    else:
        (s_ref, gap_ref, gc_in_ref, ea_ref,
         wn1_ref, bn1_ref, wn2_ref, bn2_ref, wroot_ref, bconv_ref,
         wihT_ref, whhT_ref, bih_ref, bhh_ref,
         snew_ref, msg_ref) = refs
        gc = gc_in_ref[...]
    s = s_ref[...]
    gap = gap_ref[...]
    agg = (gap[0] + gap[1]) / gc
    snew = _gru_math(s, agg, wroot_ref[...], bconv_ref[...], wihT_ref[...],
                     whhT_ref[...], bih_ref[...], bhh_ref[...])
    snew_ref[...] = snew
    msg_ref[...] = _we_msg(ea_ref[...], wn1_ref[...], bn1_ref[...],
                           wn2_ref[...], bn2_ref[...], snew)


def _edge_step(s, gap, gc_arg, ea_pad, wn1, bn1, wn2, bn2,
               wroot, bconv, wihT, whhT, bih, bhh, with_cnt):
    grid = (E_PAD // TILE_E,)
    etile = pl.BlockSpec((TILE_E, DIM), lambda i: (i, 0))
    parts = pl.BlockSpec((NC, TILE_E, DIM), lambda i: (0, i, 0))
    out_shape = [jax.ShapeDtypeStruct((E_PAD, DIM), jnp.float32),
                 jax.ShapeDtypeStruct((E_PAD, DIM), jnp.float32)]
    out_specs = [etile, etile]
    if with_cnt:
        out_shape.append(jax.ShapeDtypeStruct((E_PAD, DIM), jnp.float32))
        out_specs.append(etile)
    return pl.pallas_call(
        functools.partial(_edge_step_body, with_cnt),
        grid=grid,
        in_specs=[
            etile,
            parts,
            parts if with_cnt else etile,
            pl.BlockSpec((TILE_E, EDGE_DIM), lambda i: (i, 0)),
            pl.BlockSpec((EDGE_DIM, WIDTH), lambda i: (0, 0)),
            pl.BlockSpec((1, WIDTH), lambda i: (0, 0)),
            pl.BlockSpec((WIDTH, DIM * DIM), lambda i: (0, 0)),
            pl.BlockSpec((1, DIM * DIM), lambda i: (0, 0)),
            pl.BlockSpec((DIM, DIM), lambda i: (0, 0)),
            pl.BlockSpec((1, DIM), lambda i: (0, 0)),
            pl.BlockSpec((DIM, 3 * DIM), lambda i: (0, 0)),
            pl.BlockSpec((DIM, 3 * DIM), lambda i: (0, 0)),
            pl.BlockSpec((1, 3 * DIM), lambda i: (0, 0)),
            pl.BlockSpec((1, 3 * DIM), lambda i: (0, 0)),
        ],
        out_specs=out_specs,
        out_shape=out_shape,
    )(s, gap, gc_arg,
      ea_pad, wn1, bn1.reshape(1, WIDTH), wn2, bn2.reshape(1, DIM * DIM),
      wroot, bconv.reshape(1, DIM), wihT, whhT,
      bih.reshape(1, 3 * DIM), bhh.reshape(1, 3 * DIM))


def _gru_chain_body(out_ref, aggp0_ref, aggp1_ref, aggp2_ref, cntp_ref,
                    wroot_ref, bconv_ref, wihT_ref, whhT_ref,
                    bih_ref, bhh_ref, out3_ref):
    cntp = cntp_ref[...]
    cnt = jnp.maximum(cntp[0] + cntp[1], 1.0)
    out = out_ref[...]
    for aggp_ref in (aggp0_ref, aggp1_ref, aggp2_ref):
        aggp = aggp_ref[...]
        agg = (aggp[0] + aggp[1]) / cnt
        out = _gru_math(out, agg, wroot_ref[...], bconv_ref[...],
                        wihT_ref[...], whhT_ref[...], bih_ref[...],
                        bhh_ref[...])
    out3_ref[...] = out


TILE_N = 2000


def _gru_chain(out0, aggp0, aggp1, aggp2, cntp, wroot, bconv,
               wihT, whhT, bih, bhh):
    grid = (N // TILE_N,)
    ntile = pl.BlockSpec((TILE_N, DIM), lambda i: (i, 0))
    ptile = pl.BlockSpec((NC, TILE_N, DIM), lambda i: (0, i, 0))
    return pl.pallas_call(
        _gru_chain_body,
        grid=grid,
        in_specs=[
            ntile, ptile, ptile, ptile, ptile,
            pl.BlockSpec((DIM, DIM), lambda i: (0, 0)),
            pl.BlockSpec((1, DIM), lambda i: (0, 0)),
            pl.BlockSpec((DIM, 3 * DIM), lambda i: (0, 0)),
            pl.BlockSpec((DIM, 3 * DIM), lambda i: (0, 0)),
            pl.BlockSpec((1, 3 * DIM), lambda i: (0, 0)),
            pl.BlockSpec((1, 3 * DIM), lambda i: (0, 0)),
        ],
        out_specs=ntile,
        out_shape=jax.ShapeDtypeStruct((N, DIM), jnp.float32),
    )(out0, aggp0, aggp1, aggp2, cntp, wroot, bconv.reshape(1, DIM),
      wihT, whhT, bih.reshape(1, 3 * DIM), bhh.reshape(1, 3 * DIM))


def _set2set_body(out_ref, batch_ref, wlihT_ref, wlhhT_ref, blih_ref,
                  blhh_ref, w1_ref, b1_ref, w2_ref, b2_ref, o2_ref):
    out = out_ref[...]
    batch = batch_ref[...]
    seg_iota = lax.broadcasted_iota(jnp.int32, (N, B), 1)
    m_hot = (batch == seg_iota).astype(jnp.float32)
    neg_inf = jnp.float32(-jnp.inf)

    q_star = jnp.zeros((B, 2 * DIM), jnp.float32)
    hs = jnp.zeros((B, DIM), jnp.float32)
    cs = jnp.zeros((B, DIM), jnp.float32)
    for _ in range(3):
        g = (jnp.dot(q_star, wlihT_ref[...], preferred_element_type=jnp.float32)
             + blih_ref[...]
             + jnp.dot(hs, wlhhT_ref[...], preferred_element_type=jnp.float32)
             + blhh_ref[...])
        i_g = jax.nn.sigmoid(g[:, :DIM])
        f_g = jax.nn.sigmoid(g[:, DIM:2 * DIM])
        g_g = jnp.tanh(g[:, 2 * DIM:3 * DIM])
        o_g = jax.nn.sigmoid(g[:, 3 * DIM:])
        cs = f_g * cs + i_g * g_g
        hs = o_g * jnp.tanh(cs)
        q = hs
        qn = jnp.dot(m_hot, q, preferred_element_type=jnp.float32,
                     precision=lax.Precision.HIGHEST)
        e = jnp.sum(out * qn, axis=1, keepdims=True)
        eb = jnp.where(m_hot > 0.0, e, neg_inf)
        emax_row = jnp.max(eb, axis=0, keepdims=True)
        emax_row = jnp.where(emax_row > neg_inf, emax_row, 0.0)
        emaxg = jnp.max(jnp.where(m_hot > 0.0, emax_row, neg_inf),
                        axis=1, keepdims=True)
        e2 = jnp.exp(e - emaxg)
        denom = lax.dot_general(m_hot, e2, (((0,), (0,)), ((), ())),
                                preferred_element_type=jnp.float32,
                                precision=lax.Precision.HIGHEST)
        ru = lax.dot_general(m_hot * e2, out, (((0,), (0,)), ((), ())),
                             preferred_element_type=jnp.float32,
                             precision=lax.Precision.HIGHEST)
        r_pool = ru / jnp.maximum(denom, 1e-16)
        q_star = jnp.concatenate([q, r_pool], axis=1)

    o1 = jnp.dot(q_star, w1_ref[...], preferred_element_type=jnp.float32)
    o1 = jnp.maximum(o1 + b1_ref[...], 0.0)
    o2 = jnp.dot(o1, w2_ref[...], preferred_element_type=jnp.float32)
    o2_ref[...] = o2 + b2_ref[...]


def _set2set(out3, batch2d, wlihT, wlhhT, blih, blhh, w1, b1, w2, b2):
    return pl.pallas_call(
        _set2set_body,
        out_shape=jax.ShapeDtypeStruct((B, 1), jnp.float32),
    )(out3, batch2d,
      wlihT, wlhhT, blih.reshape(1, 4 * DIM), blhh.reshape(1, 4 * DIM),
      w1, b1.reshape(1, DIM), w2, b2.reshape(1, 1))


def _make_sc_gather():
    mesh = plsc.VectorSubcoreMesh(core_axis_name="c", subcore_axis_name="s",
                                  num_cores=NC, num_subcores=NS)

    @functools.partial(
        pl.kernel,
        out_type=jax.ShapeDtypeStruct((E_PAD, DIM), jnp.float32),
        mesh=mesh,
        compiler_params=pltpu.CompilerParams(use_tc_tiling_on_sc=False),
        scratch_types=[
            pltpu.VMEM((NCHUNK, CHUNK), jnp.int32),
            pltpu.VMEM((EPW, DIM), jnp.float32),
            pltpu.SemaphoreType.DMA,
        ],
    )
    def gather_k(table_hbm, idx_hbm, g_hbm, idx_v, rows_v, sem):
        wid = lax.axis_index("s") * NC + lax.axis_index("c")
        pltpu.sync_copy(idx_hbm.at[pl.ds(wid * NCHUNK, NCHUNK)], idx_v)

        def fire(j, carry):
            pltpu.async_copy(table_hbm.at[idx_v.at[j]],
                             rows_v.at[pl.ds(j * CHUNK, CHUNK)], sem)
            return carry

        lax.fori_loop(0, NCHUNK, fire, 0)

        def drain(j, carry):
            pltpu.make_async_copy(table_hbm.at[pl.ds(0, CHUNK)],
                                  rows_v.at[pl.ds(0, CHUNK)], sem).wait()
            return carry

        lax.fori_loop(0, NCHUNK, drain, 0)
        pltpu.sync_copy(rows_v, g_hbm.at[pl.ds(wid * EPW, EPW)])

    return gather_k


def _make_sc_scatter_gather(with_cnt, with_gather):
    mesh = plsc.VectorSubcoreMesh(core_axis_name="c", subcore_axis_name="s",
                                  num_cores=NC, num_subcores=NS)
    out_type = [jax.ShapeDtypeStruct((NC, N_PAD, DIM), jnp.float32)]
    if with_gather:
        out_type.append(jax.ShapeDtypeStruct((NC, E_PAD, DIM), jnp.float32))
    if with_cnt:
        out_type.append(jax.ShapeDtypeStruct((NC, N_PAD, DIM), jnp.float32))
        out_type.append(jax.ShapeDtypeStruct((NC, E_PAD, DIM), jnp.float32))
    scratch = [
        pltpu.VMEM((NCHUNK, CHUNK), jnp.int32),
        pltpu.VMEM((EPW, DIM), jnp.float32),
        pltpu.VMEM((ROWS_PER_TILE, DIM), jnp.float32),
        pltpu.VMEM_SHARED((N_PAD, DIM), jnp.float32),
        pltpu.SemaphoreType.DMA,
    ]
    if with_cnt:
        scratch.append(pltpu.VMEM((CHUNK, DIM), jnp.float32))
        scratch.append(pltpu.VMEM_SHARED((N_PAD, DIM), jnp.float32))

    @functools.partial(
        pl.kernel,
        out_type=tuple(out_type),
        mesh=mesh,
        compiler_params=pltpu.CompilerParams(use_tc_tiling_on_sc=False),
        scratch_types=scratch,
    )
    def scatter_k(msg_hbm, dstidx_hbm, srcidx_hbm, *refs):
        refs = list(refs)
        agg_hbm = refs.pop(0)
        ga_hbm = refs.pop(0) if with_gather else None
        if with_cnt:
            cnt_hbm = refs.pop(0)
            gc_hbm = refs.pop(0)
        idx_v = refs.pop(0)
        rows_v = refs.pop(0)
        zbuf_v = refs.pop(0)
        sh_agg = refs.pop(0)
        sem = refs.pop(0)
        if with_cnt:
            obuf_v = refs.pop(0)
            sh_cnt = refs.pop(0)
        cid = lax.axis_index("c")
        sid = lax.axis_index("s")
        wid = sid * NC + cid

        def zrow(i, carry):
            zbuf_v[i] = jnp.zeros((DIM,), jnp.float32)
            return carry

        lax.fori_loop(0, ROWS_PER_TILE, zrow, 0)
        pltpu.sync_copy(zbuf_v, sh_agg.at[pl.ds(sid * ROWS_PER_TILE,
                                                ROWS_PER_TILE)])
        if with_cnt:
            def orow(i, carry):
                obuf_v[i] = jnp.ones((DIM,), jnp.float32)
                return carry

            lax.fori_loop(0, CHUNK, orow, 0)
            pltpu.sync_copy(zbuf_v, sh_cnt.at[pl.ds(sid * ROWS_PER_TILE,
                                                    ROWS_PER_TILE)])
        plsc.subcore_barrier()

        pltpu.sync_copy(dstidx_hbm.at[pl.ds(wid * NCHUNK, NCHUNK)], idx_v)
        pltpu.sync_copy(msg_hbm.at[pl.ds(wid * EPW, EPW)], rows_v)

        def step(j, carry):
            pltpu.sync_copy(rows_v.at[pl.ds(j * CHUNK, CHUNK)],
                            sh_agg.at[idx_v.at[j]], add=True)
            if with_cnt:
                pltpu.sync_copy(obuf_v, sh_cnt.at[idx_v.at[j]], add=True)
            return carry

        lax.fori_loop(0, NCHUNK, step, 0)
        plsc.subcore_barrier()

        row0 = sid * ROWS_PER_TILE
        pltpu.sync_copy(sh_agg.at[pl.ds(row0, ROWS_PER_TILE)],
                        agg_hbm.at[cid].at[pl.ds(row0, ROWS_PER_TILE)])
        if with_cnt:
            pltpu.sync_copy(sh_cnt.at[pl.ds(row0, ROWS_PER_TILE)],
                            cnt_hbm.at[cid].at[pl.ds(row0, ROWS_PER_TILE)])
        if not with_gather:
            return
        plsc.subcore_barrier()

        def gather_all(table, out_hbm):
            for p in range(GPASS):
                edge0 = sid * EPS + p * EPW
                pltpu.sync_copy(
                    srcidx_hbm.at[pl.ds(edge0 // CHUNK, NCHUNK)], idx_v)

                def fire(j, carry):
                    pltpu.async_copy(table.at[idx_v.at[j]],
                                     rows_v.at[pl.ds(j * CHUNK, CHUNK)], sem)
                    return carry

                lax.fori_loop(0, NCHUNK, fire, 0)

                def drain(j, carry):
                    pltpu.make_async_copy(table.at[pl.ds(0, CHUNK)],
                                          rows_v.at[pl.ds(0, CHUNK)],
                                          sem).wait()
                    return carry

                lax.fori_loop(0, NCHUNK, drain, 0)
                pltpu.sync_copy(rows_v, out_hbm.at[pl.ds(edge0, EPW)])

        gather_all(agg_hbm.at[cid], ga_hbm.at[cid])
        if with_cnt:
            gather_all(cnt_hbm.at[cid], gc_hbm.at[cid])

    return scatter_k


_sc_cache = {}


def _sc_gather(table, src2d):
    fn = _sc_cache.get("gather")
    if fn is None:
        fn = _sc_cache["gather"] = _make_sc_gather()
    return fn(table, src2d)


def _sc_scatter_gather(msg, dst2d, src2d, with_cnt, with_gather):
    key = ("scatter", with_cnt, with_gather)
    fn = _sc_cache.get(key)
    if fn is None:
        fn = _sc_cache[key] = _make_sc_scatter_gather(with_cnt, with_gather)
    res = fn(msg, dst2d, src2d)
    if not isinstance(res, (tuple, list)):
        res = (res,)
    res = list(res)
    agg = res.pop(0)
    ga = res.pop(0) if with_gather else None
    cnt = res.pop(0) if with_cnt else None
    gc = res.pop(0) if with_cnt else None
    return agg, ga, cnt, gc


def kernel(x, z, edge_attr, W0, b0, Wn1, bn1, Wn2, bn2, Wroot, bconv,
           W_ih, W_hh, b_ih, b_hh, Wl_ih, Wl_hh, bl_ih, bl_hh, W1, b1, W2, b2,
           edge_index, batch):
    feats = jnp.concatenate([x, z[:, None]], axis=1)
    src = edge_index[0]
    dst = edge_index[1]
    pad = E_PAD - E
    src2d = jnp.concatenate(
        [src, jnp.zeros((pad,), jnp.int32)]).reshape(E_PAD // CHUNK, CHUNK)
    dst2d = jnp.concatenate(
        [dst, jnp.full((pad,), DUMMY_ROW, jnp.int32)]).reshape(
            E_PAD // CHUNK, CHUNK)
    ea_pad = jnp.concatenate(
        [edge_attr, jnp.zeros((pad, EDGE_DIM), jnp.float32)], axis=0)

    out0 = _node_mlp(feats, W0, b0)
    wihT = W_ih.T
    whhT = W_hh.T

    g0 = _sc_gather(out0, src2d)
    msg0 = _matvec(g0, ea_pad, Wn1, bn1, Wn2, bn2)
    aggp0, ga0, cntp, gc_parts = _sc_scatter_gather(
        msg0, dst2d, src2d, with_cnt=True, with_gather=True)
    s1, msg1, gc = _edge_step(
        g0, ga0, gc_parts, ea_pad, Wn1, bn1, Wn2, bn2,
        Wroot, bconv, wihT, whhT, b_ih, b_hh, with_cnt=True)
    aggp1, ga1, _, _ = _sc_scatter_gather(
        msg1, dst2d, src2d, with_cnt=False, with_gather=True)
    s2, msg2 = _edge_step(
        s1, ga1, gc, ea_pad, Wn1, bn1, Wn2, bn2,
        Wroot, bconv, wihT, whhT, b_ih, b_hh, with_cnt=False)
    aggp2, _, _, _ = _sc_scatter_gather(
        msg2, dst2d, src2d, with_cnt=False, with_gather=False)

    out3 = _gru_chain(out0, aggp0[:, :N, :], aggp1[:, :N, :], aggp2[:, :N, :],
                      cntp[:, :N, :], Wroot, bconv, wihT, whhT, b_ih, b_hh)
    batch2d = batch.reshape(N, 1)
    o2 = _set2set(out3, batch2d, Wl_ih.T, Wl_hh.T, bl_ih, bl_hh,
                  W1, b1, W2, b2)
    return o2.reshape(-1)


# speedup vs baseline: 1.1041x; 1.1041x over previous
import functools

import jax
import jax.numpy as jnp
from jax import lax
from jax.experimental import pallas as pl
from jax.experimental.pallas import tpu as pltpu
from jax.experimental.pallas import tpu_sc as plsc

N = 10000
E = 160000
INPUT_DIM = 128
DIM = 16
WIDTH = 64
B = 64
EDGE_DIM = 5

NC = 2
NS = 16
NW = NC * NS

EPW = 5120
E_PAD = NW * EPW
CHUNK = 128
NCHUNK = EPW // CHUNK
N_PAD = 10240
ROWS_PER_TILE = N_PAD // NS
DUMMY_ROW = N

TILE_E = 2048


def _node_mlp_body(feats_ref, w_ref, b_ref, out_ref):
    acc = jnp.dot(feats_ref[...], w_ref[...], preferred_element_type=jnp.float32)
    out_ref[...] = jnp.maximum(acc + b_ref[...], 0.0)


def _node_mlp(feats, w0, b0):
    return pl.pallas_call(
        _node_mlp_body,
        out_shape=jax.ShapeDtypeStruct((N, DIM), jnp.float32),
    )(feats, w0, b0.reshape(1, DIM))


def _we_body(ea_ref, wn1_ref, bn1_ref, wn2_ref, bn2_ref, we_ref):
    hidden = jnp.dot(ea_ref[...], wn1_ref[...], preferred_element_type=jnp.float32,
                     precision=lax.Precision.HIGHEST)
    hidden = jnp.maximum(hidden + bn1_ref[...], 0.0)
    we = jnp.dot(hidden, wn2_ref[...], preferred_element_type=jnp.float32,
                 precision=lax.Precision.HIGHEST)
    we_ref[...] = we + bn2_ref[...]


def _compute_we(ea_pad, wn1, bn1, wn2, bn2):
    grid = (E_PAD // TILE_E,)
    return pl.pallas_call(
        _we_body,
        grid=grid,
        in_specs=[
            pl.BlockSpec((TILE_E, EDGE_DIM), lambda i: (i, 0)),
            pl.BlockSpec((EDGE_DIM, WIDTH), lambda i: (0, 0)),
            pl.BlockSpec((1, WIDTH), lambda i: (0, 0)),
            pl.BlockSpec((WIDTH, DIM * DIM), lambda i: (0, 0)),
            pl.BlockSpec((1, DIM * DIM), lambda i: (0, 0)),
        ],
        out_specs=pl.BlockSpec((TILE_E, DIM * DIM), lambda i: (i, 0)),
        out_shape=jax.ShapeDtypeStruct((E_PAD, DIM * DIM), jnp.float32),
    )(ea_pad, wn1, bn1.reshape(1, WIDTH), wn2, bn2.reshape(1, DIM * DIM))


def _matvec_body(g_ref, we_ref, msg_ref):
    g = g_ref[...]
    we = we_ref[...]
    acc = g[:, 0:1] * we[:, 0:DIM]
    for d in range(1, DIM):
        acc = acc + g[:, d:d + 1] * we[:, DIM * d:DIM * (d + 1)]
    msg_ref[...] = acc


def _matvec(g, we):
    grid = (E_PAD // TILE_E,)
    return pl.pallas_call(
        _matvec_body,
        grid=grid,
        in_specs=[
            pl.BlockSpec((TILE_E, DIM), lambda i: (i, 0)),
            pl.BlockSpec((TILE_E, DIM * DIM), lambda i: (i, 0)),
        ],
        out_specs=pl.BlockSpec((TILE_E, DIM), lambda i: (i, 0)),
        out_shape=jax.ShapeDtypeStruct((E_PAD, DIM), jnp.float32),
    )(g, we)


def _gru_body(out_ref, h_ref, aggp_ref, cntp_ref, wroot_ref, bconv_ref,
              wihT_ref, whhT_ref, bih_ref, bhh_ref, hout_ref):
    out = out_ref[...]
    h = h_ref[...]
    aggp = aggp_ref[...]
    cntp = cntp_ref[...]
    cnt = jnp.maximum(cntp[0] + cntp[1], 1.0)
    agg = (aggp[0] + aggp[1]) / cnt
    m = jnp.dot(out, wroot_ref[...], preferred_element_type=jnp.float32)
    m = jnp.maximum(m + agg + bconv_ref[...], 0.0)
    gi = jnp.dot(m, wihT_ref[...], preferred_element_type=jnp.float32) + bih_ref[...]
    gh = jnp.dot(h, whhT_ref[...], preferred_element_type=jnp.float32) + bhh_ref[...]
    r = jax.nn.sigmoid(gi[:, :DIM] + gh[:, :DIM])
    u = jax.nn.sigmoid(gi[:, DIM:2 * DIM] + gh[:, DIM:2 * DIM])
    n = jnp.tanh(gi[:, 2 * DIM:] + r * gh[:, 2 * DIM:])
    hout_ref[...] = (1.0 - u) * n + u * h


def _gru(out, h, aggp, cntp, wroot, bconv, wihT, whhT, bih, bhh):
    return pl.pallas_call(
        _gru_body,
        out_shape=jax.ShapeDtypeStruct((N, DIM), jnp.float32),
    )(out, h, aggp, cntp, wroot, bconv.reshape(1, DIM),
      wihT, whhT, bih.reshape(1, 3 * DIM), bhh.reshape(1, 3 * DIM))


def _set2set_body(out_ref, batch_ref, wlihT_ref, wlhhT_ref, blih_ref, blhh_ref,
                  w1_ref, b1_ref, w2_ref, b2_ref, o2_ref):
    out = out_ref[...]
    batch = batch_ref[...]
    seg_iota = lax.broadcasted_iota(jnp.int32, (N, B), 1)
    m_hot = (batch == seg_iota).astype(jnp.float32)
    neg_inf = jnp.float32(-jnp.inf)

    q_star = jnp.zeros((B, 2 * DIM), jnp.float32)
    hs = jnp.zeros((B, DIM), jnp.float32)
    cs = jnp.zeros((B, DIM), jnp.float32)
    for _ in range(3):
        g = (jnp.dot(q_star, wlihT_ref[...], preferred_element_type=jnp.float32)
             + blih_ref[...]
             + jnp.dot(hs, wlhhT_ref[...], preferred_element_type=jnp.float32)
             + blhh_ref[...])
        i_g = jax.nn.sigmoid(g[:, :DIM])
        f_g = jax.nn.sigmoid(g[:, DIM:2 * DIM])
        g_g = jnp.tanh(g[:, 2 * DIM:3 * DIM])
        o_g = jax.nn.sigmoid(g[:, 3 * DIM:])
        cs = f_g * cs + i_g * g_g
        hs = o_g * jnp.tanh(cs)
        q = hs
        qn = jnp.dot(m_hot, q, preferred_element_type=jnp.float32,
                     precision=lax.Precision.HIGHEST)
        e = jnp.sum(out * qn, axis=1, keepdims=True)
        eb = jnp.where(m_hot > 0.0, e, neg_inf)
        emax_row = jnp.max(eb, axis=0, keepdims=True)
        emax_row = jnp.where(emax_row > neg_inf, emax_row, 0.0)
        emaxg = jnp.max(jnp.where(m_hot > 0.0, emax_row, neg_inf),
                        axis=1, keepdims=True)
        e2 = jnp.exp(e - emaxg)
        denom = lax.dot_general(m_hot, e2, (((0,), (0,)), ((), ())),
                                preferred_element_type=jnp.float32,
                                precision=lax.Precision.HIGHEST)
        ru = lax.dot_general(m_hot * e2, out, (((0,), (0,)), ((), ())),
                             preferred_element_type=jnp.float32,
                             precision=lax.Precision.HIGHEST)
        r_pool = ru / jnp.maximum(denom, 1e-16)
        q_star = jnp.concatenate([q, r_pool], axis=1)

    o1 = jnp.dot(q_star, w1_ref[...], preferred_element_type=jnp.float32)
    o1 = jnp.maximum(o1 + b1_ref[...], 0.0)
    o2 = jnp.dot(o1, w2_ref[...], preferred_element_type=jnp.float32)
    o2_ref[...] = o2 + b2_ref[...]


def _set2set(out, batch2d, wlihT, wlhhT, blih, blhh, w1, b1, w2, b2):
    return pl.pallas_call(
        _set2set_body,
        out_shape=jax.ShapeDtypeStruct((B, 1), jnp.float32),
    )(out, batch2d, wlihT, wlhhT, blih.reshape(1, 4 * DIM),
      blhh.reshape(1, 4 * DIM), w1, b1.reshape(1, DIM), w2, b2.reshape(1, 1))


def _make_sc_gather():
    mesh = plsc.VectorSubcoreMesh(core_axis_name="c", subcore_axis_name="s", num_cores=NC, num_subcores=NS)

    @functools.partial(
        pl.kernel,
        out_type=jax.ShapeDtypeStruct((E_PAD, DIM), jnp.float32),
        mesh=mesh,
        compiler_params=pltpu.CompilerParams(use_tc_tiling_on_sc=False),
        scratch_types=[
            pltpu.VMEM((NCHUNK, CHUNK), jnp.int32),
            pltpu.VMEM((EPW, DIM), jnp.float32),
            pltpu.SemaphoreType.DMA,
        ],
    )
    def gather_k(table_hbm, idx_hbm, g_hbm, idx_v, rows_v, sem):
        wid = lax.axis_index("s") * NC + lax.axis_index("c")
        pltpu.sync_copy(idx_hbm.at[pl.ds(wid * NCHUNK, NCHUNK)], idx_v)

        def fire(j, carry):
            pltpu.async_copy(table_hbm.at[idx_v.at[j]],
                             rows_v.at[pl.ds(j * CHUNK, CHUNK)], sem)
            return carry

        lax.fori_loop(0, NCHUNK, fire, 0)

        def drain(j, carry):
            pltpu.make_async_copy(table_hbm.at[pl.ds(0, CHUNK)],
                                  rows_v.at[pl.ds(0, CHUNK)], sem).wait()
            return carry

        lax.fori_loop(0, NCHUNK, drain, 0)
        pltpu.sync_copy(rows_v, g_hbm.at[pl.ds(wid * EPW, EPW)])

    return gather_k


def _make_sc_scatter(with_cnt):
    mesh = plsc.VectorSubcoreMesh(core_axis_name="c", subcore_axis_name="s", num_cores=NC, num_subcores=NS)
    out_type = [jax.ShapeDtypeStruct((NC, N_PAD, DIM), jnp.float32)]
    scratch = [
        pltpu.VMEM((NCHUNK, CHUNK), jnp.int32),
        pltpu.VMEM((EPW, DIM), jnp.float32),
        pltpu.VMEM((ROWS_PER_TILE, DIM), jnp.float32),
        pltpu.VMEM_SHARED((N_PAD, DIM), jnp.float32),
    ]
    if with_cnt:
        out_type.append(jax.ShapeDtypeStruct((NC, N_PAD, DIM), jnp.float32))
        scratch.append(pltpu.VMEM((CHUNK, DIM), jnp.float32))
        scratch.append(pltpu.VMEM_SHARED((N_PAD, DIM), jnp.float32))

    @functools.partial(
        pl.kernel,
        out_type=tuple(out_type),
        mesh=mesh,
        compiler_params=pltpu.CompilerParams(use_tc_tiling_on_sc=False),
        scratch_types=scratch,
    )
    def scatter_k(msg_hbm, idx_hbm, *refs):
        if with_cnt:
            (agg_hbm, cnt_hbm, idx_v, rows_v, zbuf_v, sh_agg,
             obuf_v, sh_cnt) = refs
        else:
            agg_hbm, idx_v, rows_v, zbuf_v, sh_agg = refs
        cid = lax.axis_index("c")
        sid = lax.axis_index("s")
        wid = sid * NC + cid

        def zrow(i, carry):
            zbuf_v[i] = jnp.zeros((DIM,), jnp.float32)
            return carry

        lax.fori_loop(0, ROWS_PER_TILE, zrow, 0)
        pltpu.sync_copy(zbuf_v, sh_agg.at[pl.ds(sid * ROWS_PER_TILE,
                                                ROWS_PER_TILE)])
        if with_cnt:
            def orow(i, carry):
                obuf_v[i] = jnp.ones((DIM,), jnp.float32)
                return carry

            lax.fori_loop(0, CHUNK, orow, 0)
            pltpu.sync_copy(zbuf_v, sh_cnt.at[pl.ds(sid * ROWS_PER_TILE,
                                                    ROWS_PER_TILE)])
        plsc.subcore_barrier()

        pltpu.sync_copy(idx_hbm.at[pl.ds(wid * NCHUNK, NCHUNK)], idx_v)
        pltpu.sync_copy(msg_hbm.at[pl.ds(wid * EPW, EPW)], rows_v)

        def step(j, carry):
            pltpu.sync_copy(rows_v.at[pl.ds(j * CHUNK, CHUNK)],
                            sh_agg.at[idx_v.at[j]], add=True)
            if with_cnt:
                pltpu.sync_copy(obuf_v, sh_cnt.at[idx_v.at[j]], add=True)
            return carry

        lax.fori_loop(0, NCHUNK, step, 0)
        plsc.subcore_barrier()

        row0 = sid * ROWS_PER_TILE
        pltpu.sync_copy(sh_agg.at[pl.ds(row0, ROWS_PER_TILE)],
                        agg_hbm.at[cid].at[pl.ds(row0, ROWS_PER_TILE)])
        if with_cnt:
            pltpu.sync_copy(sh_cnt.at[pl.ds(row0, ROWS_PER_TILE)],
                            cnt_hbm.at[cid].at[pl.ds(row0, ROWS_PER_TILE)])

    return scatter_k


_sc_cache = {}


def _sc_gather(table, src2d):
    fn = _sc_cache.get("gather")
    if fn is None:
        fn = _sc_cache["gather"] = _make_sc_gather()
    return fn(table, src2d)


def _sc_scatter(msg, dst2d, with_cnt):
    fn = _sc_cache.get(("scatter", with_cnt))
    if fn is None:
        fn = _sc_cache[("scatter", with_cnt)] = _make_sc_scatter(with_cnt)
    if with_cnt:
        return fn(msg, dst2d)
    return fn(msg, dst2d)[0], None


def kernel(x, z, edge_attr, W0, b0, Wn1, bn1, Wn2, bn2, Wroot, bconv,
           W_ih, W_hh, b_ih, b_hh, Wl_ih, Wl_hh, bl_ih, bl_hh, W1, b1, W2, b2,
           edge_index, batch):
    feats = jnp.concatenate([x, z[:, None]], axis=1)
    src = edge_index[0]
    dst = edge_index[1]
    pad = E_PAD - E
    src2d = jnp.concatenate(
        [src, jnp.zeros((pad,), jnp.int32)]).reshape(E_PAD // CHUNK, CHUNK)
    dst2d = jnp.concatenate(
        [dst, jnp.full((pad,), DUMMY_ROW, jnp.int32)]).reshape(
            E_PAD // CHUNK, CHUNK)
    ea_pad = jnp.concatenate(
        [edge_attr, jnp.zeros((pad, EDGE_DIM), jnp.float32)], axis=0)

    out = _node_mlp(feats, W0, b0)
    h = out
    we = _compute_we(ea_pad, Wn1, bn1, Wn2, bn2)

    wihT = W_ih.T
    whhT = W_hh.T
    cntp = None
    for it in range(3):
        g = _sc_gather(out, src2d)
        msg = _matvec(g, we)
        aggp, cnt_new = _sc_scatter(msg, dst2d, it == 0)
        if it == 0:
            cntp = cnt_new
        h = _gru(out, h, aggp[:, :N, :], cntp[:, :N, :],
                 Wroot, bconv, wihT, whhT, b_ih, b_hh)
        out = h

    batch2d = batch.reshape(N, 1)
    o2 = _set2set(out, batch2d, Wl_ih.T, Wl_hh.T, bl_ih, bl_hh,
                  W1, b1, W2, b2)
    return o2.reshape(-1)


# speedup vs baseline: 1.1840x; 1.0723x over previous
import functools

import jax
import jax.numpy as jnp
from jax import lax
from jax.experimental import pallas as pl
from jax.experimental.pallas import tpu as pltpu
from jax.experimental.pallas import tpu_sc as plsc

N = 10000
E = 160000
INPUT_DIM = 128
DIM = 16
WIDTH = 64
B = 64
EDGE_DIM = 5

NC = 2
NS = 16
NW = NC * NS

EPW = 5120
E_PAD = NW * EPW
CHUNK = 128
NCHUNK = EPW // CHUNK
N_PAD = 10240
ROWS_PER_TILE = N_PAD // NS
DUMMY_ROW = N

TILE_E = 2048


def _node_mlp_body(feats_ref, w_ref, b_ref, out_ref):
    acc = jnp.dot(feats_ref[...], w_ref[...], preferred_element_type=jnp.float32)
    out_ref[...] = jnp.maximum(acc + b_ref[...], 0.0)


def _node_mlp(feats, w0, b0):
    return pl.pallas_call(
        _node_mlp_body,
        out_shape=jax.ShapeDtypeStruct((N, DIM), jnp.float32),
    )(feats, w0, b0.reshape(1, DIM))


def _we_body(ea_ref, wn1_ref, bn1_ref, wn2_ref, bn2_ref, we_ref):
    hidden = jnp.dot(ea_ref[...], wn1_ref[...], preferred_element_type=jnp.float32)
    hidden = jnp.maximum(hidden + bn1_ref[...], 0.0)
    we = jnp.dot(hidden, wn2_ref[...], preferred_element_type=jnp.float32)
    we_ref[...] = we + bn2_ref[...]


def _compute_we(ea_pad, wn1, bn1, wn2, bn2):
    grid = (E_PAD // TILE_E,)
    return pl.pallas_call(
        _we_body,
        grid=grid,
        in_specs=[
            pl.BlockSpec((TILE_E, EDGE_DIM), lambda i: (i, 0)),
            pl.BlockSpec((EDGE_DIM, WIDTH), lambda i: (0, 0)),
            pl.BlockSpec((1, WIDTH), lambda i: (0, 0)),
            pl.BlockSpec((WIDTH, DIM * DIM), lambda i: (0, 0)),
            pl.BlockSpec((1, DIM * DIM), lambda i: (0, 0)),
        ],
        out_specs=pl.BlockSpec((TILE_E, DIM * DIM), lambda i: (i, 0)),
        out_shape=jax.ShapeDtypeStruct((E_PAD, DIM * DIM), jnp.float32),
    )(ea_pad, wn1, bn1.reshape(1, WIDTH), wn2, bn2.reshape(1, DIM * DIM))


def _matvec_body(g_ref, we_ref, msg_ref):
    g = g_ref[...]
    we = we_ref[...]
    acc = g[:, 0:1] * we[:, 0:DIM]
    for d in range(1, DIM):
        acc = acc + g[:, d:d + 1] * we[:, DIM * d:DIM * (d + 1)]
    msg_ref[...] = acc


def _matvec(g, we):
    grid = (E_PAD // TILE_E,)
    return pl.pallas_call(
        _matvec_body,
        grid=grid,
        in_specs=[
            pl.BlockSpec((TILE_E, DIM), lambda i: (i, 0)),
            pl.BlockSpec((TILE_E, DIM * DIM), lambda i: (i, 0)),
        ],
        out_specs=pl.BlockSpec((TILE_E, DIM), lambda i: (i, 0)),
        out_shape=jax.ShapeDtypeStruct((E_PAD, DIM), jnp.float32),
    )(g, we)


def _gru_body(out_ref, h_ref, aggp_ref, cntp_ref, wroot_ref, bconv_ref,
              wihT_ref, whhT_ref, bih_ref, bhh_ref, hout_ref):
    out = out_ref[...]
    h = h_ref[...]
    aggp = aggp_ref[...]
    cntp = cntp_ref[...]
    cnt = jnp.maximum(cntp[0] + cntp[1], 1.0)
    agg = (aggp[0] + aggp[1]) / cnt
    m = jnp.dot(out, wroot_ref[...], preferred_element_type=jnp.float32)
    m = jnp.maximum(m + agg + bconv_ref[...], 0.0)
    gi = jnp.dot(m, wihT_ref[...], preferred_element_type=jnp.float32) + bih_ref[...]
    gh = jnp.dot(h, whhT_ref[...], preferred_element_type=jnp.float32) + bhh_ref[...]
    r = jax.nn.sigmoid(gi[:, :DIM] + gh[:, :DIM])
    u = jax.nn.sigmoid(gi[:, DIM:2 * DIM] + gh[:, DIM:2 * DIM])
    n = jnp.tanh(gi[:, 2 * DIM:] + r * gh[:, 2 * DIM:])
    hout_ref[...] = (1.0 - u) * n + u * h


def _gru(out, h, aggp, cntp, wroot, bconv, wihT, whhT, bih, bhh):
    return pl.pallas_call(
        _gru_body,
        out_shape=jax.ShapeDtypeStruct((N, DIM), jnp.float32),
    )(out, h, aggp, cntp, wroot, bconv.reshape(1, DIM),
      wihT, whhT, bih.reshape(1, 3 * DIM), bhh.reshape(1, 3 * DIM))


def _set2set_body(out_ref, batch_ref, wlihT_ref, wlhhT_ref, blih_ref, blhh_ref,
                  w1_ref, b1_ref, w2_ref, b2_ref, o2_ref):
    out = out_ref[...]
    batch = batch_ref[...]
    seg_iota = lax.broadcasted_iota(jnp.int32, (N, B), 1)
    m_hot = (batch == seg_iota).astype(jnp.float32)
    neg_inf = jnp.float32(-jnp.inf)

    q_star = jnp.zeros((B, 2 * DIM), jnp.float32)
    hs = jnp.zeros((B, DIM), jnp.float32)
    cs = jnp.zeros((B, DIM), jnp.float32)
    for _ in range(3):
        g = (jnp.dot(q_star, wlihT_ref[...], preferred_element_type=jnp.float32)
             + blih_ref[...]
             + jnp.dot(hs, wlhhT_ref[...], preferred_element_type=jnp.float32)
             + blhh_ref[...])
        i_g = jax.nn.sigmoid(g[:, :DIM])
        f_g = jax.nn.sigmoid(g[:, DIM:2 * DIM])
        g_g = jnp.tanh(g[:, 2 * DIM:3 * DIM])
        o_g = jax.nn.sigmoid(g[:, 3 * DIM:])
        cs = f_g * cs + i_g * g_g
        hs = o_g * jnp.tanh(cs)
        q = hs
        qn = jnp.dot(m_hot, q, preferred_element_type=jnp.float32,
                     precision=lax.Precision.HIGHEST)
        e = jnp.sum(out * qn, axis=1, keepdims=True)
        eb = jnp.where(m_hot > 0.0, e, neg_inf)
        emax_row = jnp.max(eb, axis=0, keepdims=True)
        emax_row = jnp.where(emax_row > neg_inf, emax_row, 0.0)
        emaxg = jnp.max(jnp.where(m_hot > 0.0, emax_row, neg_inf),
                        axis=1, keepdims=True)
        e2 = jnp.exp(e - emaxg)
        denom = lax.dot_general(m_hot, e2, (((0,), (0,)), ((), ())),
                                preferred_element_type=jnp.float32,
                                precision=lax.Precision.HIGHEST)
        ru = lax.dot_general(m_hot * e2, out, (((0,), (0,)), ((), ())),
                             preferred_element_type=jnp.float32,
                             precision=lax.Precision.HIGHEST)
        r_pool = ru / jnp.maximum(denom, 1e-16)
        q_star = jnp.concatenate([q, r_pool], axis=1)

    o1 = jnp.dot(q_star, w1_ref[...], preferred_element_type=jnp.float32)
    o1 = jnp.maximum(o1 + b1_ref[...], 0.0)
    o2 = jnp.dot(o1, w2_ref[...], preferred_element_type=jnp.float32)
    o2_ref[...] = o2 + b2_ref[...]


def _set2set(out, batch2d, wlihT, wlhhT, blih, blhh, w1, b1, w2, b2):
    return pl.pallas_call(
        _set2set_body,
        out_shape=jax.ShapeDtypeStruct((B, 1), jnp.float32),
    )(out, batch2d, wlihT, wlhhT, blih.reshape(1, 4 * DIM),
      blhh.reshape(1, 4 * DIM), w1, b1.reshape(1, DIM), w2, b2.reshape(1, 1))


def _make_sc_gather():
    mesh = plsc.VectorSubcoreMesh(core_axis_name="c", subcore_axis_name="s", num_cores=NC, num_subcores=NS)

    @functools.partial(
        pl.kernel,
        out_type=jax.ShapeDtypeStruct((E_PAD, DIM), jnp.float32),
        mesh=mesh,
        compiler_params=pltpu.CompilerParams(use_tc_tiling_on_sc=False),
        scratch_types=[
            pltpu.VMEM((NCHUNK, CHUNK), jnp.int32),
            pltpu.VMEM((EPW, DIM), jnp.float32),
            pltpu.SemaphoreType.DMA,
        ],
    )
    def gather_k(table_hbm, idx_hbm, g_hbm, idx_v, rows_v, sem):
        wid = lax.axis_index("s") * NC + lax.axis_index("c")
        pltpu.sync_copy(idx_hbm.at[pl.ds(wid * NCHUNK, NCHUNK)], idx_v)

        def fire(j, carry):
            pltpu.async_copy(table_hbm.at[idx_v.at[j]],
                             rows_v.at[pl.ds(j * CHUNK, CHUNK)], sem)
            return carry

        lax.fori_loop(0, NCHUNK, fire, 0)

        def drain(j, carry):
            pltpu.make_async_copy(table_hbm.at[pl.ds(0, CHUNK)],
                                  rows_v.at[pl.ds(0, CHUNK)], sem).wait()
            return carry

        lax.fori_loop(0, NCHUNK, drain, 0)
        pltpu.sync_copy(rows_v, g_hbm.at[pl.ds(wid * EPW, EPW)])

    return gather_k


def _make_sc_scatter(with_cnt):
    mesh = plsc.VectorSubcoreMesh(core_axis_name="c", subcore_axis_name="s", num_cores=NC, num_subcores=NS)
    out_type = [jax.ShapeDtypeStruct((NC, N_PAD, DIM), jnp.float32)]
    scratch = [
        pltpu.VMEM((NCHUNK, CHUNK), jnp.int32),
        pltpu.VMEM((EPW, DIM), jnp.float32),
        pltpu.VMEM((ROWS_PER_TILE, DIM), jnp.float32),
        pltpu.VMEM_SHARED((N_PAD, DIM), jnp.float32),
    ]
    if with_cnt:
        out_type.append(jax.ShapeDtypeStruct((NC, N_PAD, DIM), jnp.float32))
        scratch.append(pltpu.VMEM((CHUNK, DIM), jnp.float32))
        scratch.append(pltpu.VMEM_SHARED((N_PAD, DIM), jnp.float32))

    @functools.partial(
        pl.kernel,
        out_type=tuple(out_type),
        mesh=mesh,
        compiler_params=pltpu.CompilerParams(use_tc_tiling_on_sc=False),
        scratch_types=scratch,
    )
    def scatter_k(msg_hbm, idx_hbm, *refs):
        if with_cnt:
            (agg_hbm, cnt_hbm, idx_v, rows_v, zbuf_v, sh_agg,
             obuf_v, sh_cnt) = refs
        else:
            agg_hbm, idx_v, rows_v, zbuf_v, sh_agg = refs
        cid = lax.axis_index("c")
        sid = lax.axis_index("s")
        wid = sid * NC + cid

        def zrow(i, carry):
            zbuf_v[i] = jnp.zeros((DIM,), jnp.float32)
            return carry

        lax.fori_loop(0, ROWS_PER_TILE, zrow, 0)
        pltpu.sync_copy(zbuf_v, sh_agg.at[pl.ds(sid * ROWS_PER_TILE,
                                                ROWS_PER_TILE)])
        if with_cnt:
            def orow(i, carry):
                obuf_v[i] = jnp.ones((DIM,), jnp.float32)
                return carry

            lax.fori_loop(0, CHUNK, orow, 0)
            pltpu.sync_copy(zbuf_v, sh_cnt.at[pl.ds(sid * ROWS_PER_TILE,
                                                    ROWS_PER_TILE)])
        plsc.subcore_barrier()

        pltpu.sync_copy(idx_hbm.at[pl.ds(wid * NCHUNK, NCHUNK)], idx_v)
        pltpu.sync_copy(msg_hbm.at[pl.ds(wid * EPW, EPW)], rows_v)

        def step(j, carry):
            pltpu.sync_copy(rows_v.at[pl.ds(j * CHUNK, CHUNK)],
                            sh_agg.at[idx_v.at[j]], add=True)
            if with_cnt:
                pltpu.sync_copy(obuf_v, sh_cnt.at[idx_v.at[j]], add=True)
            return carry

        lax.fori_loop(0, NCHUNK, step, 0)
        plsc.subcore_barrier()

        row0 = sid * ROWS_PER_TILE
        pltpu.sync_copy(sh_agg.at[pl.ds(row0, ROWS_PER_TILE)],
                        agg_hbm.at[cid].at[pl.ds(row0, ROWS_PER_TILE)])
        if with_cnt:
            pltpu.sync_copy(sh_cnt.at[pl.ds(row0, ROWS_PER_TILE)],
                            cnt_hbm.at[cid].at[pl.ds(row0, ROWS_PER_TILE)])

    return scatter_k


_sc_cache = {}


def _sc_gather(table, src2d):
    fn = _sc_cache.get("gather")
    if fn is None:
        fn = _sc_cache["gather"] = _make_sc_gather()
    return fn(table, src2d)


def _sc_scatter(msg, dst2d, with_cnt):
    fn = _sc_cache.get(("scatter", with_cnt))
    if fn is None:
        fn = _sc_cache[("scatter", with_cnt)] = _make_sc_scatter(with_cnt)
    if with_cnt:
        return fn(msg, dst2d)
    return fn(msg, dst2d)[0], None


def kernel(x, z, edge_attr, W0, b0, Wn1, bn1, Wn2, bn2, Wroot, bconv,
           W_ih, W_hh, b_ih, b_hh, Wl_ih, Wl_hh, bl_ih, bl_hh, W1, b1, W2, b2,
           edge_index, batch):
    feats = jnp.concatenate([x, z[:, None]], axis=1)
    src = edge_index[0]
    dst = edge_index[1]
    pad = E_PAD - E
    src2d = jnp.concatenate(
        [src, jnp.zeros((pad,), jnp.int32)]).reshape(E_PAD // CHUNK, CHUNK)
    dst2d = jnp.concatenate(
        [dst, jnp.full((pad,), DUMMY_ROW, jnp.int32)]).reshape(
            E_PAD // CHUNK, CHUNK)
    ea_pad = jnp.concatenate(
        [edge_attr, jnp.zeros((pad, EDGE_DIM), jnp.float32)], axis=0)

    out = _node_mlp(feats, W0, b0)
    h = out
    we = _compute_we(ea_pad, Wn1, bn1, Wn2, bn2)

    wihT = W_ih.T
    whhT = W_hh.T
    cntp = None
    for it in range(3):
        g = _sc_gather(out, src2d)
        msg = _matvec(g, we)
        aggp, cnt_new = _sc_scatter(msg, dst2d, it == 0)
        if it == 0:
            cntp = cnt_new
        h = _gru(out, h, aggp[:, :N, :], cntp[:, :N, :],
                 Wroot, bconv, wihT, whhT, b_ih, b_hh)
        out = h

    batch2d = batch.reshape(N, 1)
    o2 = _set2set(out, batch2d, Wl_ih.T, Wl_hh.T, bl_ih, bl_hh,
                  W1, b1, W2, b2)
    return o2.reshape(-1)
